# baseline (speedup 1.0000x reference)
import jax
import jax.numpy as jnp
from jax import lax
from jax.experimental import pallas as pl
from jax.experimental.pallas import tpu as pltpu
from jax.experimental.pallas import tpu_sc as plsc

NW = 32
VOCAB = 1000000
D = 64
NFULL = 7812
SEQ = 50
BATCH = 16384
NBLK = (BATCH // 128) * SEQ


def _relayout_body(tt_ref, r_ref, bin_ref, bout_ref, gsem, osem):
    wid = lax.axis_index("s") * 2 + lax.axis_index("c")
    iota = lax.iota(jnp.int32, 16)
    dvec = [iota + 16 * m for m in range(4)]

    n_k = (NFULL - wid + 31) // 32

    def transpose_cols(ncols):
        def tbody(i, carry):
            for u in range(4):
                v_loc = 4 * i + u
                cvec = jnp.full((16,), v_loc, jnp.int32)
                for m in range(4):
                    bout_ref[v_loc, pl.ds(16 * m, 16)] = plsc.load_gather(
                        bin_ref, [dvec[m], cvec])
            return carry
        lax.fori_loop(0, ncols // 4, tbody, 0)

    def body(k, carry):
        vt = wid + 32 * k
        copies = []
        for dt in range(8):
            copies.append(pltpu.async_copy(
                tt_ref.at[pl.ds(8 * dt, 8), pl.ds(vt * 128, 128)],
                bin_ref.at[pl.ds(8 * dt, 8), pl.ds(0, 128)], gsem))
        for c in copies:
            c.wait()
        transpose_cols(128)
        pltpu.sync_copy(bout_ref, r_ref.at[pl.ds(vt * 128, 128), :])
        return carry

    lax.fori_loop(0, n_k, body, 0)

    @pl.when(wid == NW - 1)
    def _tail():
        copies = []
        for d in range(D):
            copies.append(pltpu.async_copy(
                tt_ref.at[d, pl.ds(NFULL * 128, 64)],
                bin_ref.at[d, pl.ds(0, 64)], osem))
        for c in copies:
            c.wait()
        transpose_cols(64)
        pltpu.sync_copy(
            bout_ref.at[pl.ds(0, 64), :],
            r_ref.at[pl.ds(NFULL * 128, 64), :])


def _gather_body(r_ref, idx_ref, out_ref, idxv_ref, staged_ref, obuf_ref,
                 gsem, osem):
    wid = lax.axis_index("s") * 2 + lax.axis_index("c")
    iota = lax.iota(jnp.int32, 16)
    dvec = [iota + 16 * m for m in range(4)]
    per_w = NBLK // NW

    def body(k, carry):
        blk = wid * per_w + k
        s = blk // 128
        bt = blk - s * 128
        pltpu.sync_copy(idx_ref.at[blk], idxv_ref)
        pltpu.async_copy(r_ref.at[idxv_ref], staged_ref, gsem).wait()
        def tbody(i, carry):
            for u in range(4):
                bc = 4 * i + u
                cvec = jnp.full((16,), bc, jnp.int32)
                for m in range(4):
                    plsc.store_scatter(
                        obuf_ref, [dvec[m], cvec],
                        staged_ref[bc, pl.ds(16 * m, 16)])
            return carry
        lax.fori_loop(0, 32, tbody, 0)
        for dt in range(8):
            pltpu.sync_copy(
                obuf_ref.at[pl.ds(8 * dt, 8), pl.ds(0, 128)],
                out_ref.at[s, dt, bt])
        return carry

    lax.fori_loop(0, per_w, body, 0)


def kernel(data, table):
    mesh = plsc.VectorSubcoreMesh(core_axis_name="c", subcore_axis_name="s")
    params = pltpu.CompilerParams(use_tc_tiling_on_sc=True,
                                  needs_layout_passes=False)

    r = pl.kernel(
        _relayout_body,
        out_type=jax.ShapeDtypeStruct((VOCAB, 128), jnp.float32),
        mesh=mesh,
        compiler_params=params,
        scratch_types=[
            pltpu.VMEM((64, 129), jnp.float32),
            pltpu.VMEM((128, 128), jnp.float32),
            pltpu.SemaphoreType.DMA,
            pltpu.SemaphoreType.DMA,
        ],
    )(table.T)

    idx5 = data.T.reshape(NBLK, 128)
    out5 = pl.kernel(
        _gather_body,
        out_type=jax.ShapeDtypeStruct((SEQ, 8, 128, 8, 128), jnp.float32),
        mesh=mesh,
        compiler_params=params,
        scratch_types=[
            pltpu.VMEM((128,), jnp.int32),
            pltpu.VMEM((128, 128), jnp.float32),
            pltpu.VMEM((64, 129), jnp.float32),
            pltpu.SemaphoreType.DMA,
            pltpu.SemaphoreType.DMA,
        ],
    )(r, idx5)

    return out5.transpose(2, 4, 0, 1, 3).reshape(BATCH, SEQ, D)


# speedup vs baseline: 1.7871x; 1.7871x over previous
import jax
import jax.numpy as jnp
from jax import lax
from jax.experimental import pallas as pl
from jax.experimental.pallas import tpu as pltpu
from jax.experimental.pallas import tpu_sc as plsc

NW = 32
VOCAB = 1000000
D = 64
NFULL = 7812
SEQ = 50
BATCH = 16384
NBLK = (BATCH // 128) * SEQ
PAD = 137


def _relayout_body(tt_ref, r_ref, bin0, bin1, bout0, bout1,
                   gsem0, gsem1, tsem):
    wid = lax.axis_index("s") * 2 + lax.axis_index("c")
    iota = lax.iota(jnp.int32, 16)
    dvec = [iota + 16 * m for m in range(4)]
    bins = (bin0, bin1)
    bouts = (bout0, bout1)
    gsems = (gsem0, gsem1)
    n_pairs = 123

    def fire_in(k, b):
        vt = wid + 32 * k
        @pl.when(vt < NFULL)
        def _():
            for dt in range(8):
                pltpu.async_copy(
                    tt_ref.at[pl.ds(8 * dt, 8), pl.ds(vt * 128, 128)],
                    bins[b].at[pl.ds(8 * dt, 8), pl.ds(0, 128)], gsems[b])

    def wait_in(k, b):
        vt = wid + 32 * k
        @pl.when(vt < NFULL)
        def _():
            for dt in range(8):
                pltpu.make_async_copy(
                    tt_ref.at[pl.ds(8 * dt, 8), pl.ds(vt * 128, 128)],
                    bins[b].at[pl.ds(8 * dt, 8), pl.ds(0, 128)],
                    gsems[b]).wait()

    def transpose_out(k, b):
        vt = wid + 32 * k
        @pl.when(vt < NFULL)
        def _():
            @plsc.parallel_loop(0, 128, step=1, unroll=8)
            def tbody(v_loc):
                cvec = jnp.full((16,), 0, jnp.int32) + v_loc
                for m in range(4):
                    bouts[b][v_loc, pl.ds(16 * m, 16)] = plsc.load_gather(
                        bins[b], [dvec[m], cvec])
            pltpu.sync_copy(bouts[b], r_ref.at[pl.ds(vt * 128, 128), :])

    fire_in(0, 0)

    def body(i, carry):
        for b in range(2):
            k = 2 * i + b
            fire_in(k + 1, 1 - b)
            wait_in(k, b)
            transpose_out(k, b)
        return carry

    lax.fori_loop(0, n_pairs, body, 0)

    @pl.when(wid == NW - 1)
    def _tail():
        copies = []
        for d in range(D):
            copies.append(pltpu.async_copy(
                tt_ref.at[d, pl.ds(NFULL * 128, 64)],
                bin0.at[d, pl.ds(0, 64)], tsem))
        for c in copies:
            c.wait()

        @plsc.parallel_loop(0, 64, step=1, unroll=8)
        def tbody(v_loc):
            cvec = jnp.full((16,), 0, jnp.int32) + v_loc
            for m in range(4):
                bout0[v_loc, pl.ds(16 * m, 16)] = plsc.load_gather(
                    bin0, [dvec[m], cvec])
        pltpu.sync_copy(
            bout0.at[pl.ds(0, 64), :],
            r_ref.at[pl.ds(NFULL * 128, 64), :])


def _gather_body(r_ref, idx_ref, out_ref, idxv0, idxv1, staged0, staged1,
                 obuf_ref, gsem0, gsem1):
    wid = lax.axis_index("s") * 2 + lax.axis_index("c")
    iota = lax.iota(jnp.int32, 16)
    dvec = [iota + 16 * m for m in range(4)]
    idxvs = (idxv0, idxv1)
    stageds = (staged0, staged1)
    gsems = (gsem0, gsem1)
    per_w = NBLK // NW

    def fire(k, b):
        @pl.when(k < per_w)
        def _():
            blk = wid * per_w + k
            pltpu.sync_copy(idx_ref.at[blk], idxvs[b])
            pltpu.async_copy(r_ref.at[idxvs[b]], stageds[b], gsems[b])

    def wait(k, b):
        pltpu.make_async_copy(
            r_ref.at[idxvs[b]], stageds[b], gsems[b]).wait()

    fire(0, 0)

    def body(i, carry):
        for b in range(2):
            k = 2 * i + b
            fire(k + 1, 1 - b)
            wait(k, b)
            blk = wid * per_w + k
            s = blk // 128
            bt = blk - s * 128

            @plsc.parallel_loop(0, 128, step=1, unroll=8)
            def tbody(bc):
                cvec = jnp.full((16,), 0, jnp.int32) + bc
                for m in range(4):
                    plsc.store_scatter(
                        obuf_ref, [dvec[m], cvec],
                        stageds[b][bc, pl.ds(16 * m, 16)])
            for dt in range(8):
                pltpu.sync_copy(
                    obuf_ref.at[pl.ds(8 * dt, 8), pl.ds(0, 128)],
                    out_ref.at[s, dt, bt])
        return carry

    lax.fori_loop(0, per_w // 2, body, 0)


def kernel(data, table):
    mesh = plsc.VectorSubcoreMesh(core_axis_name="c", subcore_axis_name="s")
    params = pltpu.CompilerParams(use_tc_tiling_on_sc=True,
                                  needs_layout_passes=False)

    r = pl.kernel(
        _relayout_body,
        out_type=jax.ShapeDtypeStruct((VOCAB, 128), jnp.float32),
        mesh=mesh,
        compiler_params=params,
        scratch_types=[
            pltpu.VMEM((64, PAD), jnp.float32),
            pltpu.VMEM((64, PAD), jnp.float32),
            pltpu.VMEM((128, 128), jnp.float32),
            pltpu.VMEM((128, 128), jnp.float32),
            pltpu.SemaphoreType.DMA,
            pltpu.SemaphoreType.DMA,
            pltpu.SemaphoreType.DMA,
        ],
    )(table.T)

    idx5 = data.T.reshape(NBLK, 128)
    out5 = pl.kernel(
        _gather_body,
        out_type=jax.ShapeDtypeStruct((SEQ, 8, 128, 8, 128), jnp.float32),
        mesh=mesh,
        compiler_params=params,
        scratch_types=[
            pltpu.VMEM((128,), jnp.int32),
            pltpu.VMEM((128,), jnp.int32),
            pltpu.VMEM((128, 128), jnp.float32),
            pltpu.VMEM((128, 128), jnp.float32),
            pltpu.VMEM((64, PAD), jnp.float32),
            pltpu.SemaphoreType.DMA,
            pltpu.SemaphoreType.DMA,
        ],
    )(r, idx5)

    return out5.transpose(2, 4, 0, 1, 3).reshape(BATCH, SEQ, D)


# speedup vs baseline: 2.1163x; 1.1842x over previous
import jax
import jax.numpy as jnp
from jax import lax
from jax.experimental import pallas as pl
from jax.experimental.pallas import tpu as pltpu
from jax.experimental.pallas import tpu_sc as plsc

NW = 32
VOCAB = 1000000
D = 64
NFULL = 7812
SEQ = 50
BATCH = 16384
NBLK = (BATCH // 128) * SEQ
PAD = 137


def _relayout_body(tt_ref, r_ref, bin0, bin1, bout0, bout1,
                   gsem0, gsem1, osem0, osem1, tsem):
    wid = lax.axis_index("s") * 2 + lax.axis_index("c")
    iota = lax.iota(jnp.int32, 16)
    dvec = [iota + 16 * m for m in range(4)]
    bins = (bin0, bin1)
    bouts = (bout0, bout1)
    gsems = (gsem0, gsem1)
    osems = (osem0, osem1)
    n_pairs = 123

    def fire_in(k, b):
        vt = wid + 32 * k
        @pl.when(vt < NFULL)
        def _():
            for dt in range(8):
                pltpu.async_copy(
                    tt_ref.at[pl.ds(8 * dt, 8), pl.ds(vt * 128, 128)],
                    bins[b].at[pl.ds(8 * dt, 8), pl.ds(0, 128)], gsems[b])

    def wait_in(k, b):
        vt = wid + 32 * k
        @pl.when(vt < NFULL)
        def _():
            for dt in range(8):
                pltpu.make_async_copy(
                    tt_ref.at[pl.ds(8 * dt, 8), pl.ds(vt * 128, 128)],
                    bins[b].at[pl.ds(8 * dt, 8), pl.ds(0, 128)],
                    gsems[b]).wait()

    def drain_out(k, b):
        vt = wid + 32 * k
        @pl.when((k >= 0) & (vt < NFULL))
        def _():
            pltpu.make_async_copy(
                bouts[b], r_ref.at[pl.ds(vt * 128, 128), :],
                osems[b]).wait()

    def transpose_out(k, b):
        vt = wid + 32 * k
        @pl.when(vt < NFULL)
        def _():
            @plsc.parallel_loop(0, 128, step=1, unroll=8)
            def tbody(v_loc):
                cvec = jnp.full((16,), 0, jnp.int32) + v_loc
                for m in range(4):
                    bouts[b][v_loc, pl.ds(16 * m, 16)] = plsc.load_gather(
                        bins[b], [dvec[m], cvec])
            pltpu.async_copy(
                bouts[b], r_ref.at[pl.ds(vt * 128, 128), :], osems[b])

    fire_in(0, 0)

    def body(i, carry):
        for b in range(2):
            k = 2 * i + b
            fire_in(k + 1, 1 - b)
            wait_in(k, b)
            drain_out(k - 2, b)
            transpose_out(k, b)
        return carry

    lax.fori_loop(0, n_pairs, body, 0)
    drain_out(244, 0)
    drain_out(245, 1)

    @pl.when(wid == NW - 1)
    def _tail():
        copies = []
        for d in range(D):
            copies.append(pltpu.async_copy(
                tt_ref.at[d, pl.ds(NFULL * 128, 64)],
                bin0.at[d, pl.ds(0, 64)], tsem))
        for c in copies:
            c.wait()

        @plsc.parallel_loop(0, 64, step=1, unroll=8)
        def tbody(v_loc):
            cvec = jnp.full((16,), 0, jnp.int32) + v_loc
            for m in range(4):
                bout0[v_loc, pl.ds(16 * m, 16)] = plsc.load_gather(
                    bin0, [dvec[m], cvec])
        pltpu.sync_copy(
            bout0.at[pl.ds(0, 64), :],
            r_ref.at[pl.ds(NFULL * 128, 64), :])


def _gather_body(r_ref, idx_ref, out_ref, idxv0, idxv1, staged0, staged1,
                 obuf0, obuf1, gsem0, gsem1, osem0, osem1):
    wid = lax.axis_index("s") * 2 + lax.axis_index("c")
    iota = lax.iota(jnp.int32, 16)
    dvec = [iota + 16 * m for m in range(4)]
    idxvs = (idxv0, idxv1)
    stageds = (staged0, staged1)
    obufs = (obuf0, obuf1)
    gsems = (gsem0, gsem1)
    osems = (osem0, osem1)
    per_w = NBLK // NW

    def fire(k, b):
        @pl.when(k < per_w)
        def _():
            blk = wid * per_w + k
            pltpu.sync_copy(idx_ref.at[blk], idxvs[b])
            pltpu.async_copy(r_ref.at[idxvs[b]], stageds[b], gsems[b])

    def wait(k, b):
        pltpu.make_async_copy(
            r_ref.at[idxvs[b]], stageds[b], gsems[b]).wait()

    def s_bt(k):
        blk = wid * per_w + k
        s = blk // 128
        return s, blk - s * 128

    def drain_out(k, b):
        @pl.when(k >= 0)
        def _():
            s, bt = s_bt(k)
            for dt in range(8):
                pltpu.make_async_copy(
                    obufs[b].at[pl.ds(8 * dt, 8), pl.ds(0, 128)],
                    out_ref.at[s, dt, bt], osems[b]).wait()

    fire(0, 0)

    def body(i, carry):
        for b in range(2):
            k = 2 * i + b
            fire(k + 1, 1 - b)
            wait(k, b)
            drain_out(k - 2, b)
            s, bt = s_bt(k)

            @plsc.parallel_loop(0, 128, step=1, unroll=8)
            def tbody(bc):
                cvec = jnp.full((16,), 0, jnp.int32) + bc
                for m in range(4):
                    plsc.store_scatter(
                        obufs[b], [dvec[m], cvec],
                        stageds[b][bc, pl.ds(16 * m, 16)])
            for dt in range(8):
                pltpu.async_copy(
                    obufs[b].at[pl.ds(8 * dt, 8), pl.ds(0, 128)],
                    out_ref.at[s, dt, bt], osems[b])
        return carry

    lax.fori_loop(0, per_w // 2, body, 0)
    drain_out(per_w - 2, 0)
    drain_out(per_w - 1, 1)


def kernel(data, table):
    mesh = plsc.VectorSubcoreMesh(core_axis_name="c", subcore_axis_name="s")
    params = pltpu.CompilerParams(use_tc_tiling_on_sc=True,
                                  needs_layout_passes=False)

    r = pl.kernel(
        _relayout_body,
        out_type=jax.ShapeDtypeStruct((VOCAB, 128), jnp.float32),
        mesh=mesh,
        compiler_params=params,
        scratch_types=[
            pltpu.VMEM((64, PAD), jnp.float32),
            pltpu.VMEM((64, PAD), jnp.float32),
            pltpu.VMEM((128, 128), jnp.float32),
            pltpu.VMEM((128, 128), jnp.float32),
            pltpu.SemaphoreType.DMA,
            pltpu.SemaphoreType.DMA,
            pltpu.SemaphoreType.DMA,
            pltpu.SemaphoreType.DMA,
            pltpu.SemaphoreType.DMA,
        ],
    )(table.T)

    idx5 = data.T.reshape(NBLK, 128)
    out5 = pl.kernel(
        _gather_body,
        out_type=jax.ShapeDtypeStruct((SEQ, 8, 128, 8, 128), jnp.float32),
        mesh=mesh,
        compiler_params=params,
        scratch_types=[
            pltpu.VMEM((128,), jnp.int32),
            pltpu.VMEM((128,), jnp.int32),
            pltpu.VMEM((128, 128), jnp.float32),
            pltpu.VMEM((128, 128), jnp.float32),
            pltpu.VMEM((64, PAD), jnp.float32),
            pltpu.VMEM((64, PAD), jnp.float32),
            pltpu.SemaphoreType.DMA,
            pltpu.SemaphoreType.DMA,
            pltpu.SemaphoreType.DMA,
            pltpu.SemaphoreType.DMA,
        ],
    )(r, idx5)

    return out5.transpose(2, 4, 0, 1, 3).reshape(BATCH, SEQ, D)


# speedup vs baseline: 2.2483x; 1.0624x over previous
import jax
import jax.numpy as jnp
from jax import lax
from jax.experimental import pallas as pl
from jax.experimental.pallas import tpu as pltpu
from jax.experimental.pallas import tpu_sc as plsc

NW = 32
VOCAB = 1000000
D = 64
NFULL = 7812
SEQ = 50
BATCH = 16384
NBLK = (BATCH // 128) * SEQ
PAD = 137


def _relayout_body(tt_ref, r_ref, bin0, bin1, bout0, bout1,
                   gsem0, gsem1, osem0, osem1, tsem):
    wid = lax.axis_index("s") * 2 + lax.axis_index("c")
    iota = lax.iota(jnp.int32, 16)
    dvec = [iota + 16 * m for m in range(4)]
    bins = (bin0, bin1)
    bouts = (bout0, bout1)
    gsems = (gsem0, gsem1)
    osems = (osem0, osem1)
    n_pairs = 123

    def fire_in(k, b):
        vt = wid + 32 * k
        @pl.when(vt < NFULL)
        def _():
            for dt in range(8):
                pltpu.async_copy(
                    tt_ref.at[pl.ds(8 * dt, 8), pl.ds(vt * 128, 128)],
                    bins[b].at[pl.ds(8 * dt, 8), pl.ds(0, 128)], gsems[b])

    def wait_in(k, b):
        vt = wid + 32 * k
        @pl.when(vt < NFULL)
        def _():
            for dt in range(8):
                pltpu.make_async_copy(
                    tt_ref.at[pl.ds(8 * dt, 8), pl.ds(vt * 128, 128)],
                    bins[b].at[pl.ds(8 * dt, 8), pl.ds(0, 128)],
                    gsems[b]).wait()

    def drain_out(k, b):
        vt = wid + 32 * k
        @pl.when((k >= 0) & (vt < NFULL))
        def _():
            pltpu.make_async_copy(
                bouts[b], r_ref.at[pl.ds(vt * 128, 128), :],
                osems[b]).wait()

    def transpose_out(k, b):
        vt = wid + 32 * k
        @pl.when(vt < NFULL)
        def _():
            @plsc.parallel_loop(0, 128, step=1, unroll=16)
            def tbody(v_loc):
                cvec = jnp.full((16,), 0, jnp.int32) + v_loc
                for m in range(4):
                    bouts[b][v_loc, pl.ds(16 * m, 16)] = plsc.load_gather(
                        bins[b], [dvec[m], cvec])
            pltpu.async_copy(
                bouts[b], r_ref.at[pl.ds(vt * 128, 128), :], osems[b])

    fire_in(0, 0)

    def body(i, carry):
        for b in range(2):
            k = 2 * i + b
            fire_in(k + 1, 1 - b)
            wait_in(k, b)
            drain_out(k - 2, b)
            transpose_out(k, b)
        return carry

    lax.fori_loop(0, n_pairs, body, 0)
    drain_out(244, 0)
    drain_out(245, 1)

    @pl.when(wid == NW - 1)
    def _tail():
        copies = []
        for d in range(D):
            copies.append(pltpu.async_copy(
                tt_ref.at[d, pl.ds(NFULL * 128, 64)],
                bin0.at[d, pl.ds(0, 64)], tsem))
        for c in copies:
            c.wait()

        @plsc.parallel_loop(0, 64, step=1, unroll=16)
        def tbody(v_loc):
            cvec = jnp.full((16,), 0, jnp.int32) + v_loc
            for m in range(4):
                bout0[v_loc, pl.ds(16 * m, 16)] = plsc.load_gather(
                    bin0, [dvec[m], cvec])
        pltpu.sync_copy(
            bout0.at[pl.ds(0, 64), :],
            r_ref.at[pl.ds(NFULL * 128, 64), :])


def _gather_body(r_ref, idx_ref, out_ref, idxa_ref, staged0, staged1,
                 obuf0, obuf1, gsem0, gsem1, osem0, osem1):
    wid = lax.axis_index("s") * 2 + lax.axis_index("c")
    iota = lax.iota(jnp.int32, 16)
    dvec = [iota + 16 * m for m in range(4)]
    stageds = (staged0, staged1)
    obufs = (obuf0, obuf1)
    gsems = (gsem0, gsem1)
    osems = (osem0, osem1)
    per_w = NBLK // NW

    pltpu.sync_copy(idx_ref.at[pl.ds(wid * per_w, per_w)], idxa_ref)

    def fire(k, b):
        @pl.when(k < per_w)
        def _():
            pltpu.async_copy(r_ref.at[idxa_ref.at[k]], stageds[b], gsems[b])

    def wait(k, b):
        pltpu.make_async_copy(
            r_ref.at[idxa_ref.at[k]], stageds[b], gsems[b]).wait()

    def s_bt(k):
        blk = wid * per_w + k
        s = blk // 128
        return s, blk - s * 128

    def drain_out(k, b):
        @pl.when(k >= 0)
        def _():
            s, bt = s_bt(k)
            for dt in range(8):
                pltpu.make_async_copy(
                    obufs[b].at[pl.ds(8 * dt, 8), pl.ds(0, 128)],
                    out_ref.at[s, dt, bt], osems[b]).wait()

    fire(0, 0)

    def body(i, carry):
        for b in range(2):
            k = 2 * i + b
            fire(k + 1, 1 - b)
            wait(k, b)
            drain_out(k - 2, b)
            s, bt = s_bt(k)

            @plsc.parallel_loop(0, 128, step=1, unroll=16)
            def tbody(bc):
                cvec = jnp.full((16,), 0, jnp.int32) + bc
                for m in range(4):
                    plsc.store_scatter(
                        obufs[b], [dvec[m], cvec],
                        stageds[b][bc, pl.ds(16 * m, 16)])
            for dt in range(8):
                pltpu.async_copy(
                    obufs[b].at[pl.ds(8 * dt, 8), pl.ds(0, 128)],
                    out_ref.at[s, dt, bt], osems[b])
        return carry

    lax.fori_loop(0, per_w // 2, body, 0)
    drain_out(per_w - 2, 0)
    drain_out(per_w - 1, 1)


def kernel(data, table):
    mesh = plsc.VectorSubcoreMesh(core_axis_name="c", subcore_axis_name="s")
    params = pltpu.CompilerParams(use_tc_tiling_on_sc=True,
                                  needs_layout_passes=False)

    r = pl.kernel(
        _relayout_body,
        out_type=jax.ShapeDtypeStruct((VOCAB, 128), jnp.float32),
        mesh=mesh,
        compiler_params=params,
        scratch_types=[
            pltpu.VMEM((64, PAD), jnp.float32),
            pltpu.VMEM((64, PAD), jnp.float32),
            pltpu.VMEM((128, 128), jnp.float32),
            pltpu.VMEM((128, 128), jnp.float32),
            pltpu.SemaphoreType.DMA,
            pltpu.SemaphoreType.DMA,
            pltpu.SemaphoreType.DMA,
            pltpu.SemaphoreType.DMA,
            pltpu.SemaphoreType.DMA,
        ],
    )(table.T)

    idx5 = data.T.reshape(NBLK, 128)
    out5 = pl.kernel(
        _gather_body,
        out_type=jax.ShapeDtypeStruct((SEQ, 8, 128, 8, 128), jnp.float32),
        mesh=mesh,
        compiler_params=params,
        scratch_types=[
            pltpu.VMEM((NBLK // NW, 128), jnp.int32),
            pltpu.VMEM((128, 128), jnp.float32),
            pltpu.VMEM((128, 128), jnp.float32),
            pltpu.VMEM((64, PAD), jnp.float32),
            pltpu.VMEM((64, PAD), jnp.float32),
            pltpu.SemaphoreType.DMA,
            pltpu.SemaphoreType.DMA,
            pltpu.SemaphoreType.DMA,
            pltpu.SemaphoreType.DMA,
        ],
    )(r, idx5)

    return out5.transpose(2, 4, 0, 1, 3).reshape(BATCH, SEQ, D)


# speedup vs baseline: 2.7204x; 1.2100x over previous
import jax
import jax.numpy as jnp
from jax import lax
from jax.experimental import pallas as pl
from jax.experimental.pallas import tpu as pltpu
from jax.experimental.pallas import tpu_sc as plsc

D_MODEL = 64
LANES = 128
G = 4
NBUF = 2


def _gather_body(table_hbm, idx_hbm, out_hbm, idx_v, rows_v,
                 gsem0, gsem1, osem0, osem1):
    nc = 2
    wid = lax.axis_index("s") * nc + lax.axis_index("c")
    n_chunks = idx_v.shape[0]
    n_groups = n_chunks // G
    base_chunk = wid * n_chunks
    base_row = base_chunk * LANES
    rows_per_group = G * LANES
    gsems = (gsem0, gsem1)
    osems = (osem0, osem1)

    pltpu.sync_copy(idx_hbm.at[pl.ds(base_chunk, n_chunks)], idx_v)

    def fire_gather(g, b):
        for j in range(G):
            pltpu.async_copy(
                table_hbm.at[idx_v.at[g * G + j]],
                rows_v.at[b].at[pl.ds(j * LANES, LANES)],
                gsems[b])

    def wait_gather(b):
        for j in range(G):
            pltpu.make_async_copy(
                table_hbm.at[idx_v.at[j]],
                rows_v.at[b].at[pl.ds(j * LANES, LANES)],
                gsems[b]).wait()

    def fire_out(g, b):
        pltpu.async_copy(
            rows_v.at[b],
            out_hbm.at[pl.ds(base_row + g * rows_per_group, rows_per_group)],
            osems[b])

    def wait_out(g, b):
        pltpu.make_async_copy(
            rows_v.at[b],
            out_hbm.at[pl.ds(base_row + g * rows_per_group, rows_per_group)],
            osems[b]).wait()

    fire_gather(0, 0)
    fire_gather(1, 1)

    def body(i, carry):
        for b in range(NBUF):
            g = NBUF * i + b
            wait_gather(b)
            fire_out(g, b)
            wait_out(g, b)
            fire_gather(g + NBUF, b)
        return carry

    lax.fori_loop(0, n_groups // NBUF - 1, body, 0)
    for b in range(NBUF):
        g = n_groups - NBUF + b
        wait_gather(b)
        fire_out(g, b)
        wait_out(g, b)


def kernel(data, table):
    s0, s1 = data.shape
    b = s0 * s1
    info = plsc.get_sparse_core_info()
    nw = info.num_cores * info.num_subcores
    n_chunks_total = b // LANES
    per_w = n_chunks_total // nw
    idx2d = data.reshape(n_chunks_total, LANES)

    mesh = plsc.VectorSubcoreMesh(core_axis_name="c", subcore_axis_name="s")
    out = pl.kernel(
        _gather_body,
        out_type=jax.ShapeDtypeStruct((b, D_MODEL), jnp.float32),
        mesh=mesh,
        compiler_params=pltpu.CompilerParams(use_tc_tiling_on_sc=False),
        scratch_types=[
            pltpu.VMEM((per_w, LANES), jnp.int32),
            pltpu.VMEM((NBUF, G * LANES, D_MODEL), jnp.float32),
            pltpu.SemaphoreType.DMA,
            pltpu.SemaphoreType.DMA,
            pltpu.SemaphoreType.DMA,
            pltpu.SemaphoreType.DMA,
        ],
    )(table, idx2d)
    return out.reshape(s0, s1, D_MODEL)


# speedup vs baseline: 2.7415x; 1.0078x over previous
import jax
import jax.numpy as jnp
from jax import lax
from jax.experimental import pallas as pl
from jax.experimental.pallas import tpu as pltpu
from jax.experimental.pallas import tpu_sc as plsc

D_MODEL = 64
LANES = 128
G = 5
NBUF = 2


def _gather_body(table_hbm, idx_hbm, out_hbm, idx_v, rows_v,
                 gsem0, gsem1, osem0, osem1):
    nc = 2
    wid = lax.axis_index("s") * nc + lax.axis_index("c")
    n_chunks = idx_v.shape[0]
    n_groups = n_chunks // G
    base_chunk = wid * n_chunks
    base_row = base_chunk * LANES
    rows_per_group = G * LANES
    gsems = (gsem0, gsem1)
    osems = (osem0, osem1)

    pltpu.sync_copy(idx_hbm.at[pl.ds(base_chunk, n_chunks)], idx_v)

    def fire_gather(g, b):
        for j in range(G):
            pltpu.async_copy(
                table_hbm.at[idx_v.at[g * G + j]],
                rows_v.at[b].at[pl.ds(j * LANES, LANES)],
                gsems[b])

    def wait_gather(b):
        for j in range(G):
            pltpu.make_async_copy(
                table_hbm.at[idx_v.at[j]],
                rows_v.at[b].at[pl.ds(j * LANES, LANES)],
                gsems[b]).wait()

    def fire_out(g, b):
        pltpu.async_copy(
            rows_v.at[b],
            out_hbm.at[pl.ds(base_row + g * rows_per_group, rows_per_group)],
            osems[b])

    def wait_out(g, b):
        pltpu.make_async_copy(
            rows_v.at[b],
            out_hbm.at[pl.ds(base_row + g * rows_per_group, rows_per_group)],
            osems[b]).wait()

    fire_gather(0, 0)
    fire_gather(1, 1)

    def body(i, carry):
        for b in range(NBUF):
            g = NBUF * i + b
            wait_gather(b)
            fire_out(g, b)
            wait_out(g, b)
            fire_gather(g + NBUF, b)
        return carry

    lax.fori_loop(0, n_groups // NBUF - 1, body, 0)
    for b in range(NBUF):
        g = n_groups - NBUF + b
        wait_gather(b)
        fire_out(g, b)
        wait_out(g, b)


def kernel(data, table):
    s0, s1 = data.shape
    b = s0 * s1
    info = plsc.get_sparse_core_info()
    nw = info.num_cores * info.num_subcores
    n_chunks_total = b // LANES
    per_w = n_chunks_total // nw
    idx2d = data.reshape(n_chunks_total, LANES)

    mesh = plsc.VectorSubcoreMesh(core_axis_name="c", subcore_axis_name="s")
    out = pl.kernel(
        _gather_body,
        out_type=jax.ShapeDtypeStruct((b, D_MODEL), jnp.float32),
        mesh=mesh,
        compiler_params=pltpu.CompilerParams(use_tc_tiling_on_sc=False),
        scratch_types=[
            pltpu.VMEM((per_w, LANES), jnp.int32),
            pltpu.VMEM((NBUF, G * LANES, D_MODEL), jnp.float32),
            pltpu.SemaphoreType.DMA,
            pltpu.SemaphoreType.DMA,
            pltpu.SemaphoreType.DMA,
            pltpu.SemaphoreType.DMA,
        ],
    )(table, idx2d)
    return out.reshape(s0, s1, D_MODEL)
